# baseline (speedup 1.0000x reference)
import functools

import jax
import jax.numpy as jnp
from jax.experimental import pallas as pl
from jax.experimental.pallas import tpu as pltpu

_TOPK = 8
_NEG = float("-inf")
_BIGI = 2**30
_LANES = 128


def _extract_topk(s, idx, n):
    q, w = s.shape
    pos_iota = jax.lax.broadcasted_iota(jnp.int32, (q, w), 1)
    vals, idxs = [], []
    for _ in range(n):
        m = jnp.max(s, axis=1, keepdims=True)
        pos = jnp.min(jnp.where(s == m, pos_iota, _BIGI), axis=1, keepdims=True)
        hit = pos_iota == pos
        vals.append(m)
        idxs.append(jnp.sum(jnp.where(hit, idx, 0), axis=1, keepdims=True))
        s = jnp.where(hit, _NEG, s)
    return jnp.concatenate(vals, axis=1), jnp.concatenate(idxs, axis=1)


def _get_col(s, j):
    q, b = s.shape
    cfull, rem = divmod(b, _LANES)
    if j < cfull:
        return s[:, j * _LANES:(j + 1) * _LANES]
    return jnp.concatenate(
        [s[:, cfull * _LANES:],
         jnp.full((q, _LANES - rem), _NEG, jnp.float32)], axis=1)


def _insert(ms, as_, cand, acand):
    d = len(ms)
    g = [cand > m for m in ms]
    new_ms, new_as = list(ms), list(as_)
    for k in range(d - 1, 0, -1):
        new_ms[k] = jnp.where(g[k - 1], ms[k - 1], jnp.where(g[k], cand, ms[k]))
        new_as[k] = jnp.where(g[k - 1], as_[k - 1],
                              jnp.where(g[k], acand, as_[k]))
    new_ms[0] = jnp.where(g[0], cand, ms[0])
    new_as[0] = jnp.where(g[0], acand, as_[0])
    return new_ms, new_as


def _lane_sweep(s, depth, use_pairs):
    q, b = s.shape
    ncols = -(-b // _LANES)
    neg = jnp.full((q, _LANES), _NEG, jnp.float32)
    zero = jnp.zeros((q, _LANES), jnp.int32)
    ms = [neg] * depth
    as_ = [zero] * depth
    loser_bound = neg
    if use_pairs:
        for p in range(-(-ncols // 2)):
            ca = _get_col(s, 2 * p)
            cb = (_get_col(s, 2 * p + 1) if 2 * p + 1 < ncols else neg)
            ge = ca >= cb
            w = jnp.where(ge, ca, cb)
            wcol = jnp.where(ge, 2 * p, 2 * p + 1)
            loser_bound = jnp.maximum(loser_bound, jnp.where(ge, cb, ca))
            ms, as_ = _insert(ms, as_, w, wcol)
    else:
        for j in range(ncols):
            ms, as_ = _insert(ms, as_, _get_col(s, j), j)
    return ms, as_, loser_bound


def _merge_rounds(ms, as_, base, topk):
    q = ms[0].shape[0]
    depth = len(ms)
    lane = jax.lax.broadcasted_iota(jnp.int32, (q, _LANES), 1)
    sent = -1 - lane
    levels_v = list(ms[1:]) + [jnp.full((q, _LANES), _NEG, jnp.float32)]
    levels_i = [base + a * _LANES + lane for a in as_[1:]] + [sent]
    mc = ms[0]
    ic = base + as_[0] * _LANES + lane
    vals, idxs = [], []
    for _ in range(topk):
        m = jnp.max(mc, axis=1, keepdims=True)
        pos = jnp.min(jnp.where(mc == m, ic, _BIGI), axis=1, keepdims=True)
        vals.append(m)
        idxs.append(pos)
        hit = ic == pos
        mc = jnp.where(hit, levels_v[0], mc)
        ic = jnp.where(hit, levels_i[0], ic)
        for k in range(depth - 1):
            levels_v[k] = jnp.where(hit, levels_v[k + 1], levels_v[k])
            levels_i[k] = jnp.where(hit, levels_i[k + 1], levels_i[k])
        levels_v[depth - 1] = jnp.where(hit, _NEG, levels_v[depth - 1])
        levels_i[depth - 1] = jnp.where(hit, sent, levels_i[depth - 1])
    return jnp.concatenate(vals, axis=1), jnp.concatenate(idxs, axis=1)


def _merge_running(rv, ri, bv, bi, topk):
    return _extract_topk(jnp.concatenate([rv, bv], axis=1),
                         jnp.concatenate([ri, bi], axis=1), topk)


def _topk_kernel(q_ref, k_ref, vals_ref, idx_ref, qn_ref, rv_ref, ri_ref,
                 *, block, topk, kvalid):
    i = pl.program_id(0)
    nb = pl.num_programs(0)
    qdim = q_ref.shape[0]

    @pl.when(i == 0)
    def _init():
        q = q_ref[...]
        qn2 = jnp.sum(q * q, axis=1, keepdims=True)
        qn_ref[...] = q / jnp.maximum(jnp.sqrt(qn2), 1e-12)
        rv_ref[...] = jnp.full((qdim, topk), _NEG, jnp.float32)
        ri_ref[...] = -1 - jax.lax.broadcasted_iota(jnp.int32, (qdim, topk), 1)

    kb = k_ref[...]
    kn2 = jnp.sum(kb * kb, axis=1, keepdims=True)
    kn = kb / jnp.maximum(jnp.sqrt(kn2), 1e-12)
    s = jax.lax.dot_general(qn_ref[...], kn, (((1,), (1,)), ((), ())),
                            preferred_element_type=jnp.float32)

    base = i * block
    if kvalid % block != 0:
        gidx = base + jax.lax.broadcasted_iota(jnp.int32, s.shape, 1)
        s = jnp.where(gidx < kvalid, s, _NEG)
    rv, ri = rv_ref[...], ri_ref[...]

    ms, as_, loser_bound = _lane_sweep(s, 4, use_pairs=True)
    bv, bi = _merge_rounds(ms, as_, base, topk)
    nv, ni = _merge_running(rv, ri, bv, bi, topk)
    v8 = nv[:, topk - 1:topk]
    deep = jnp.maximum(ms[3], loser_bound)
    trigger = jnp.max(jnp.where(deep >= v8, 1, 0)) > 0

    def _fallback():
        fms, fas, _ = _lane_sweep(s, topk, use_pairs=False)
        fbv, fbi = _merge_rounds(fms, fas, base, topk)
        return _merge_running(rv, ri, fbv, fbi, topk)

    nv, ni = jax.lax.cond(trigger, _fallback, lambda: (nv, ni))
    rv_ref[...] = nv
    ri_ref[...] = ni

    @pl.when(i == nb - 1)
    def _emit():
        vals_ref[...] = rv_ref[...]
        idx_ref[...] = ri_ref[...]


def _pick_block(k):
    for b in (5000, 4096, 4000, 2048, 2000, 1600, 1024, 1000, 800, 512, 400,
              256, 200, 128, 8):
        if k % b == 0 and b % 8 == 0:
            return b
    return None


def kernel(queries, keys, top_k):
    del top_k
    qdim, d = queries.shape
    k = keys.shape[0]
    block = _pick_block(k)
    kvalid = k
    if block is None:
        block = 4096
        pad = (-k) % block
        keys = jnp.pad(keys, ((0, pad), (0, 0)), constant_values=0.0)
        k = k + pad
    nb = k // block

    body = functools.partial(_topk_kernel, block=block, topk=_TOPK,
                             kvalid=kvalid)
    vals, idx = pl.pallas_call(
        body,
        grid=(nb,),
        in_specs=[
            pl.BlockSpec((qdim, d), lambda i: (0, 0)),
            pl.BlockSpec((block, d), lambda i: (i, 0)),
        ],
        out_specs=[
            pl.BlockSpec((qdim, _TOPK), lambda i: (0, 0)),
            pl.BlockSpec((qdim, _TOPK), lambda i: (0, 0)),
        ],
        out_shape=[
            jax.ShapeDtypeStruct((qdim, _TOPK), jnp.float32),
            jax.ShapeDtypeStruct((qdim, _TOPK), jnp.int32),
        ],
        scratch_shapes=[
            pltpu.VMEM((qdim, d), jnp.float32),
            pltpu.VMEM((qdim, _TOPK), jnp.float32),
            pltpu.VMEM((qdim, _TOPK), jnp.int32),
        ],
        compiler_params=pltpu.CompilerParams(
            vmem_limit_bytes=64 * 1024 * 1024),
    )(queries, keys)
    return vals, idx


# speedup vs baseline: 1.9687x; 1.9687x over previous
import functools

import jax
import jax.numpy as jnp
from jax.experimental import pallas as pl
from jax.experimental.pallas import tpu as pltpu

_TOPK = 8
_NEG = float("-inf")
_BIGI = 2**30
_LANES = 128
_QT = 256


def _extract_topk(s, idx, n):
    q, w = s.shape
    pos_iota = jax.lax.broadcasted_iota(jnp.int32, (q, w), 1)
    vals, idxs = [], []
    for _ in range(n):
        m = jnp.max(s, axis=1, keepdims=True)
        pos = jnp.min(jnp.where(s == m, pos_iota, _BIGI), axis=1, keepdims=True)
        hit = pos_iota == pos
        vals.append(m)
        idxs.append(jnp.sum(jnp.where(hit, idx, 0), axis=1, keepdims=True))
        s = jnp.where(hit, _NEG, s)
    return jnp.concatenate(vals, axis=1), jnp.concatenate(idxs, axis=1)


def _get_col(s, j):
    q, b = s.shape
    cfull, rem = divmod(b, _LANES)
    if j < cfull:
        return s[:, j * _LANES:(j + 1) * _LANES]
    return jnp.concatenate(
        [s[:, cfull * _LANES:],
         jnp.full((q, _LANES - rem), _NEG, jnp.float32)], axis=1)


def _lane_sweep(s, depth):
    q, b = s.shape
    ncols = -(-b // _LANES)
    neg = jnp.full((q, _LANES), _NEG, jnp.float32)
    zero = jnp.zeros((q, _LANES), jnp.int32)
    ms = [neg] * depth
    as_ = [zero] * depth
    for j in range(ncols):
        cand = _get_col(s, j)
        g = [cand > m for m in ms]
        new_ms, new_as = list(ms), list(as_)
        for k in range(depth - 1, 0, -1):
            new_ms[k] = jnp.where(g[k - 1], ms[k - 1],
                                  jnp.where(g[k], cand, ms[k]))
            new_as[k] = jnp.where(g[k - 1], as_[k - 1],
                                  jnp.where(g[k], j, as_[k]))
        new_ms[0] = jnp.where(g[0], cand, ms[0])
        new_as[0] = jnp.where(g[0], j, as_[0])
        ms, as_ = new_ms, new_as
    return ms, as_


def _merge_rounds(ms, as_, base, topk):
    q = ms[0].shape[0]
    depth = len(ms)
    lane = jax.lax.broadcasted_iota(jnp.int32, (q, _LANES), 1)
    sent = -1 - lane
    levels_v = list(ms[1:]) + [jnp.full((q, _LANES), _NEG, jnp.float32)]
    levels_i = [base + a * _LANES + lane for a in as_[1:]] + [sent]
    mc = ms[0]
    ic = base + as_[0] * _LANES + lane
    vals, idxs = [], []
    for _ in range(topk):
        m = jnp.max(mc, axis=1, keepdims=True)
        pos = jnp.min(jnp.where(mc == m, ic, _BIGI), axis=1, keepdims=True)
        vals.append(m)
        idxs.append(pos)
        hit = ic == pos
        mc = jnp.where(hit, levels_v[0], mc)
        ic = jnp.where(hit, levels_i[0], ic)
        for k in range(depth - 1):
            levels_v[k] = jnp.where(hit, levels_v[k + 1], levels_v[k])
            levels_i[k] = jnp.where(hit, levels_i[k + 1], levels_i[k])
        levels_v[depth - 1] = jnp.where(hit, _NEG, levels_v[depth - 1])
        levels_i[depth - 1] = jnp.where(hit, sent, levels_i[depth - 1])
    return jnp.concatenate(vals, axis=1), jnp.concatenate(idxs, axis=1)


def _merge_running(rv, ri, bv, bi, topk):
    return _extract_topk(jnp.concatenate([rv, bv], axis=1),
                         jnp.concatenate([ri, bi], axis=1), topk)


def _topk_kernel(q_ref, k_ref, vals_ref, idx_ref, kn_ref, rv_ref, ri_ref,
                 *, block, topk, kvalid, qt):
    i = pl.program_id(0)
    nb = pl.num_programs(0)
    j = pl.program_id(1)

    @pl.when(j == 0)
    def _norm_keys():
        kb = k_ref[...]
        kn2 = jnp.sum(kb * kb, axis=1, keepdims=True)
        kn_ref[...] = kb / jnp.maximum(jnp.sqrt(kn2), 1e-12)

    q = q_ref[...]
    qn2 = jnp.sum(q * q, axis=1, keepdims=True)
    qn = q / jnp.maximum(jnp.sqrt(qn2), 1e-12)
    s = jax.lax.dot_general(qn, kn_ref[...], (((1,), (1,)), ((), ())),
                            preferred_element_type=jnp.float32)

    base = i * block
    if kvalid % block != 0:
        gidx = base + jax.lax.broadcasted_iota(jnp.int32, s.shape, 1)
        s = jnp.where(gidx < kvalid, s, _NEG)

    row = j * qt
    rv0 = rv_ref[pl.ds(row, qt), :]
    ri0 = ri_ref[pl.ds(row, qt), :]
    fresh = i == 0
    rv = jnp.where(fresh, _NEG, rv0)
    ri = jnp.where(fresh,
                   -1 - jax.lax.broadcasted_iota(jnp.int32, (qt, topk), 1),
                   ri0)

    ms, as_ = _lane_sweep(s, 4)
    bv, bi = _merge_rounds(ms, as_, base, topk)
    nv, ni = _merge_running(rv, ri, bv, bi, topk)
    v8 = nv[:, topk - 1:topk]
    trigger = jnp.max(jnp.where(ms[3] >= v8, 1, 0)) > 0

    def _fallback():
        fms, fas = _lane_sweep(s, topk)
        fbv, fbi = _merge_rounds(fms, fas, base, topk)
        return _merge_running(rv, ri, fbv, fbi, topk)

    nv, ni = jax.lax.cond(trigger, _fallback, lambda: (nv, ni))
    rv_ref[pl.ds(row, qt), :] = nv
    ri_ref[pl.ds(row, qt), :] = ni

    @pl.when(i == nb - 1)
    def _emit():
        vals_ref[...] = nv
        idx_ref[...] = ni


def _pick_block(k):
    for b in (10000, 8192, 8000, 5000, 4096, 4000, 2048, 2000, 1600, 1024,
              1000, 800, 512, 400, 256, 200, 128, 8):
        if k % b == 0 and b % 8 == 0:
            return b
    return None


def kernel(queries, keys, top_k):
    del top_k
    qdim, d = queries.shape
    k = keys.shape[0]
    block = _pick_block(k)
    kvalid = k
    if block is None:
        block = 4096
        pad = (-k) % block
        keys = jnp.pad(keys, ((0, pad), (0, 0)), constant_values=0.0)
        k = k + pad
    nb = k // block
    qt = _QT if qdim % _QT == 0 else qdim
    nq = qdim // qt

    body = functools.partial(_topk_kernel, block=block, topk=_TOPK,
                             kvalid=kvalid, qt=qt)
    vals, idx = pl.pallas_call(
        body,
        grid=(nb, nq),
        in_specs=[
            pl.BlockSpec((qt, d), lambda i, j: (j, 0)),
            pl.BlockSpec((block, d), lambda i, j: (i, 0)),
        ],
        out_specs=[
            pl.BlockSpec((qt, _TOPK), lambda i, j: (j, 0)),
            pl.BlockSpec((qt, _TOPK), lambda i, j: (j, 0)),
        ],
        out_shape=[
            jax.ShapeDtypeStruct((qdim, _TOPK), jnp.float32),
            jax.ShapeDtypeStruct((qdim, _TOPK), jnp.int32),
        ],
        scratch_shapes=[
            pltpu.VMEM((block, d), jnp.float32),
            pltpu.VMEM((qdim, _TOPK), jnp.float32),
            pltpu.VMEM((qdim, _TOPK), jnp.int32),
        ],
        compiler_params=pltpu.CompilerParams(
            vmem_limit_bytes=64 * 1024 * 1024),
    )(queries, keys)
    return vals, idx
